# baseline (speedup 1.0000x reference)
import functools

import jax
import jax.numpy as jnp
from jax import lax
from jax.experimental import pallas as pl
from jax.experimental.pallas import tpu as pltpu
from jax.experimental.pallas import tpu_sc as plsc

D_MODEL = 64
SCALE = 8.0
NC, NS, L = 2, 16, 16
NW = NC * NS

XR, XC = 4096, 50
B = XR * XC
OW = 128
OR = B * D_MODEL // OW
XRPW = XR // NW
BPW = B // NW
CXR = 4
CH = CXR * XC
COR = CH // 2
NCH = XRPW // CXR
NBUF = 4

_mesh = plsc.VectorSubcoreMesh(core_axis_name="c", subcore_axis_name="s")


@functools.partial(
    pl.kernel,
    out_type=jax.ShapeDtypeStruct((OR, OW), jnp.float32),
    mesh=_mesh,
    scratch_types=[
        pltpu.VMEM((BPW,), jnp.int32),
        pltpu.VMEM((NBUF, CH, D_MODEL), jnp.float32),
        pltpu.VMEM((NBUF, COR, OW), jnp.float32),
        pltpu.SemaphoreType.DMA((NBUF,)),
        pltpu.SemaphoreType.DMA((NBUF,)),
    ],
    compiler_params=pltpu.CompilerParams(use_tc_tiling_on_sc=False),
)
def _emb_lookup(x_hbm, table_hbm, out_hbm, idx_v, rows_v, obuf, gsem, ssem):
    wid = lax.axis_index("s") * NC + lax.axis_index("c")
    base = wid * BPW
    xrbase = wid * XRPW
    pltpu.sync_copy(x_hbm.at[pl.ds(base, BPW)], idx_v)

    def start_gather(g, b):
        pltpu.async_copy(
            table_hbm.at[idx_v.at[pl.ds(g * CH, CH)]], rows_v.at[b],
            gsem.at[b])

    def wait_gather(b):
        pltpu.make_async_copy(
            table_hbm.at[idx_v.at[pl.ds(0, CH)]], rows_v.at[b],
            gsem.at[b]).wait()

    orbase = wid * (XRPW * XC * D_MODEL // OW)

    def start_scatter(g, b):
        pltpu.async_copy(
            obuf.at[b], out_hbm.at[pl.ds(orbase + g * COR, COR)],
            ssem.at[b])

    def wait_scatter(b):
        pltpu.make_async_copy(
            obuf.at[b], out_hbm.at[pl.ds(orbase, COR)],
            ssem.at[b]).wait()

    for b in range(NBUF - 1):
        start_gather(b, b)

    @pl.loop(0, NCH, step=NBUF)
    def _group(g0):
        for j in range(NBUF):
            g = g0 + j
            wait_gather(j)

            @pl.when(g >= NBUF)
            def _():
                wait_scatter(j)
            rv = rows_v.at[j]
            ov = obuf.at[j]

            @pl.loop(0, COR, unroll=4)
            def _row(rr):
                for half in range(2):
                    for c in range(D_MODEL // L):
                        ov[rr, pl.ds(half * D_MODEL + c * L, L)] = (
                            rv[2 * rr + half, pl.ds(c * L, L)] * SCALE)

            start_scatter(g, j)
            h = g + NBUF - 1
            bh = (j + NBUF - 1) % NBUF

            @pl.when(h < NCH)
            def _():
                start_gather(h, bh)

    for b in range(NBUF):
        wait_scatter(b)


def kernel(x, table):
    out2 = _emb_lookup(x.reshape(-1), table)
    return out2.reshape(XR, XC, D_MODEL)


_ = OR


# speedup vs baseline: 1.3743x; 1.3743x over previous
import functools

import jax
import jax.numpy as jnp
from jax import lax
from jax.experimental import pallas as pl
from jax.experimental.pallas import tpu as pltpu
from jax.experimental.pallas import tpu_sc as plsc

D_MODEL = 64
SCALE = 8.0
NC, NS, L = 2, 16, 16
NW = NC * NS

XR, XC = 4096, 50
B = XR * XC
OW = 128
OR = B * D_MODEL // OW
XRPW = XR // NW
BPW = B // NW
CXR = 4
CH = CXR * XC
COR = CH // 2
NCH = XRPW // CXR
NBUF = 4

_mesh = plsc.VectorSubcoreMesh(core_axis_name="c", subcore_axis_name="s")


@functools.partial(
    pl.kernel,
    out_type=jax.ShapeDtypeStruct((XR, XC * D_MODEL), jnp.float32),
    mesh=_mesh,
    scratch_types=[
        pltpu.VMEM((BPW,), jnp.int32),
        pltpu.VMEM((NBUF, CH, D_MODEL), jnp.float32),
        pltpu.VMEM((NBUF, CXR, XC * D_MODEL), jnp.float32),
        pltpu.SemaphoreType.DMA((NBUF,)),
        pltpu.SemaphoreType.DMA((NBUF,)),
    ],
    compiler_params=pltpu.CompilerParams(use_tc_tiling_on_sc=False),
)
def _emb_lookup(x_hbm, table_hbm, out_hbm, idx_v, rows_v, obuf, gsem, ssem):
    wid = lax.axis_index("s") * NC + lax.axis_index("c")
    base = wid * BPW
    xrbase = wid * XRPW
    pltpu.sync_copy(x_hbm.at[pl.ds(base, BPW)], idx_v)

    def start_gather(g, b):
        pltpu.async_copy(
            table_hbm.at[idx_v.at[pl.ds(g * CH, CH)]], rows_v.at[b],
            gsem.at[b])

    def wait_gather(b):
        pltpu.make_async_copy(
            table_hbm.at[idx_v.at[pl.ds(0, CH)]], rows_v.at[b],
            gsem.at[b]).wait()

    def start_scatter(g, b):
        pltpu.async_copy(
            obuf.at[b], out_hbm.at[pl.ds(xrbase + g * CXR, CXR)],
            ssem.at[b])

    def wait_scatter(b):
        pltpu.make_async_copy(
            obuf.at[b], out_hbm.at[pl.ds(xrbase, CXR)],
            ssem.at[b]).wait()

    for b in range(NBUF - 1):
        start_gather(b, b)

    @pl.loop(0, NCH, step=NBUF)
    def _group(g0):
        for j in range(NBUF):
            g = g0 + j
            wait_gather(j)

            @pl.when(g >= NBUF)
            def _():
                wait_scatter(j)
            rv = rows_v.at[j]
            for xr in range(CXR):
                ov = obuf.at[j].at[xr]

                @plsc.parallel_loop(0, XC, unroll=5)
                def _row(r):
                    for c in range(D_MODEL // L):
                        ov[pl.ds(r * D_MODEL + c * L, L)] = (
                            rv[xr * XC + r, pl.ds(c * L, L)] * SCALE)

            start_scatter(g, j)
            h = g + NBUF - 1
            bh = (j + NBUF - 1) % NBUF

            @pl.when(h < NCH)
            def _():
                start_gather(h, bh)

    for b in range(NBUF):
        wait_scatter(b)


def kernel(x, table):
    out2 = _emb_lookup(x.reshape(-1), table)
    return out2.reshape(XR, XC, D_MODEL)


_ = OR
